# Initial kernel scaffold; baseline (speedup 1.0000x reference)
import jax
import jax.numpy as jnp
from jax.experimental import pallas as pl


def kernel(x, weight):
    raise NotImplementedError("write your pallas kernel here")


# speedup vs baseline: 1.8463x; 1.8463x over previous
import functools

import jax
import jax.numpy as jnp
from jax import lax
from jax.experimental import pallas as pl
from jax.experimental.pallas import tpu as pltpu
from jax.experimental.pallas import tpu_sc as plsc

B = 16384 * 50
D = 64
NC = 2
NS = 16
NW = NC * NS
B_PER_W = B // NW
CHUNK = 1024
N_CHUNKS = B_PER_W // CHUNK

_mesh = plsc.VectorSubcoreMesh(core_axis_name="c", subcore_axis_name="s")


@functools.partial(
    pl.kernel,
    mesh=_mesh,
    out_type=jax.ShapeDtypeStruct((B, D), jnp.float32),
    scratch_types=[
        pltpu.VMEM((CHUNK,), jnp.int32),
        pltpu.VMEM((CHUNK, D), jnp.float32),
        pltpu.SemaphoreType.DMA,
    ],
    compiler_params=pltpu.CompilerParams(use_tc_tiling_on_sc=False),
)
def _gather(idx_hbm, table_hbm, out_hbm, idx_v, rows_v, sem):
    wid = lax.axis_index("s") * NC + lax.axis_index("c")
    base = wid * B_PER_W

    def body(i, _):
        off = base + i * CHUNK
        pltpu.sync_copy(idx_hbm.at[pl.ds(off, CHUNK)], idx_v)
        pltpu.async_copy(table_hbm.at[idx_v], rows_v, sem).wait()
        pltpu.sync_copy(rows_v, out_hbm.at[pl.ds(off, CHUNK)])
        return 0

    lax.fori_loop(0, N_CHUNKS, body, 0)


def kernel(x, weight):
    xf = x.reshape(-1).astype(jnp.int32)
    out = _gather(xf, weight)
    return out.reshape(x.shape + (weight.shape[1],))


# speedup vs baseline: 1.8748x; 1.0155x over previous
import functools

import jax
import jax.numpy as jnp
from jax import lax
from jax.experimental import pallas as pl
from jax.experimental.pallas import tpu as pltpu
from jax.experimental.pallas import tpu_sc as plsc

B = 16384 * 50
D = 64
NC = 2
NS = 16
NW = NC * NS
B_PER_W = B // NW
CHUNK = 320
N_CHUNKS = B_PER_W // CHUNK
NBUF = 4
LOOK = 2

_mesh = plsc.VectorSubcoreMesh(core_axis_name="c", subcore_axis_name="s")


@functools.partial(
    pl.kernel,
    mesh=_mesh,
    out_type=jax.ShapeDtypeStruct((B, D), jnp.float32),
    scratch_types=[
        pltpu.VMEM((N_CHUNKS, CHUNK), jnp.int32),
        *[pltpu.VMEM((CHUNK, D), jnp.float32) for _ in range(NBUF)],
        pltpu.SemaphoreType.DMA((NBUF,)),
        pltpu.SemaphoreType.DMA((NBUF,)),
    ],
    compiler_params=pltpu.CompilerParams(use_tc_tiling_on_sc=False),
)
def _gather(idx_hbm, table_hbm, out_hbm, idx_v, r0, r1, r2, r3, gsem, ssem):
    rows = [r0, r1, r2, r3]
    wid = lax.axis_index("s") * NC + lax.axis_index("c")
    base = wid * B_PER_W

    pltpu.sync_copy(idx_hbm.at[wid], idx_v)

    for b in range(LOOK):
        pltpu.async_copy(table_hbm.at[idx_v.at[b]], rows[b], gsem.at[b])

    @pl.loop(0, N_CHUNKS, step=NBUF)
    def _(j):
        for b in range(NBUF):
            g = j + b
            bn = (b + LOOK) % NBUF

            @pl.when(g + LOOK < N_CHUNKS)
            def _():
                @pl.when(g - (NBUF - LOOK) >= 0)
                def _():
                    pltpu.make_async_copy(
                        rows[bn],
                        out_hbm.at[pl.ds(base + (g + LOOK - NBUF) * CHUNK, CHUNK)],
                        ssem.at[bn],
                    ).wait()

                pltpu.async_copy(
                    table_hbm.at[idx_v.at[g + LOOK]], rows[bn], gsem.at[bn]
                )

            pltpu.make_async_copy(
                table_hbm.at[idx_v.at[g]], rows[b], gsem.at[b]
            ).wait()
            pltpu.async_copy(
                rows[b], out_hbm.at[pl.ds(base + g * CHUNK, CHUNK)], ssem.at[b]
            )

    for k in range(NBUF):
        c = N_CHUNKS - NBUF + k
        b = c % NBUF
        pltpu.make_async_copy(
            rows[b], out_hbm.at[pl.ds(base + c * CHUNK, CHUNK)], ssem.at[b]
        ).wait()


def kernel(x, weight):
    xf = x.reshape(NW, N_CHUNKS, CHUNK).astype(jnp.int32)
    out = _gather(xf, weight)
    return out.reshape(x.shape + (weight.shape[1],))
